# baseline (speedup 1.0000x reference)
import functools

import jax
import jax.numpy as jnp
from jax import lax
from jax.experimental import pallas as pl
from jax.experimental.pallas import tpu as pltpu
from jax.experimental.pallas import tpu_sc as plsc

N = 10000
E = 160000
IN_DIM = 256
HID = 256
H0 = 8
NUM_CLASSES = 128
NEG_SLOPE = 0.2

ROW_BLK = 1000
N_CHUNKS = 16
CHUNK = 128

NC = 2
NS = 16
L = 16
NW = NC * NS
EB = 64
NB = E // EB
RS = 624


def _sc_mesh():
    return plsc.VectorSubcoreMesh(core_axis_name="c", subcore_axis_name="s")


def _worker_id():
    sid = lax.axis_index("s")
    cid = lax.axis_index("c")
    return cid, sid, sid * NC + cid


def _batch_range(wid, nb_total, nw):
    per = nb_total // nw
    extra = nb_total % nw
    base = wid * per + jnp.minimum(wid, extra)
    cnt = per + (wid < extra).astype(jnp.int32)
    return base, cnt


def _zero_rows(ref, n_rows, width):
    def body(i, _):
        for j in range(width // L):
            ref[i, pl.ds(j * L, L)] = jnp.zeros((L,), jnp.float32)
        return 0
    lax.fori_loop(0, n_rows, body, 0)


def _zero_spmem(sp_ref, zb, sid):
    r0 = sid * RS

    def body(k, _):
        pltpu.sync_copy(zb, sp_ref.at[pl.ds(pl.multiple_of(r0 + k * 16, 16), 16)])
        return 0
    lax.fori_loop(0, RS // 16 + (sid == NS - 1).astype(jnp.int32), body, 0)


def _copyout_spmem(sp_ref, out_ref, sid):
    r0 = pl.multiple_of(sid * RS, 16)
    pltpu.sync_copy(sp_ref.at[pl.ds(r0, RS)], out_ref.at[pl.ds(r0, RS)])

    @pl.when(sid == NS - 1)
    def _():
        pltpu.sync_copy(sp_ref.at[pl.ds(NS * RS, N - NS * RS)],
                        out_ref.at[pl.ds(NS * RS, N - NS * RS)])


def _projA_body(x_ref, w_ref, welp_ref, werp_ref, feat_ref, elt_ref, ert_ref):
    x = x_ref[...]
    feat_ref[...] = jnp.dot(x, w_ref[0], preferred_element_type=jnp.float32)
    elt_ref[...] = jnp.dot(x, welp_ref[...], preferred_element_type=jnp.float32)
    ert_ref[...] = jnp.dot(x, werp_ref[...], preferred_element_type=jnp.float32)


def _projA(x, Wcm, welp, werp):
    grid = (N // ROW_BLK, N_CHUNKS)
    return pl.pallas_call(
        _projA_body,
        grid=grid,
        in_specs=[
            pl.BlockSpec((ROW_BLK, IN_DIM), lambda i, c: (i, 0)),
            pl.BlockSpec((1, IN_DIM, CHUNK), lambda i, c: (c, 0, 0)),
            pl.BlockSpec((IN_DIM, 128), lambda i, c: (0, 0)),
            pl.BlockSpec((IN_DIM, 128), lambda i, c: (0, 0)),
        ],
        out_specs=[
            pl.BlockSpec((ROW_BLK, CHUNK),
                         lambda i, c: (c * (N // ROW_BLK) + i, 0)),
            pl.BlockSpec((ROW_BLK, 128), lambda i, c: (i, 0)),
            pl.BlockSpec((ROW_BLK, 128), lambda i, c: (i, 0)),
        ],
        out_shape=[
            jax.ShapeDtypeStruct((N_CHUNKS * N, CHUNK), jnp.float32),
            jax.ShapeDtypeStruct((N, 128), jnp.float32),
            jax.ShapeDtypeStruct((N, 128), jnp.float32),
        ],
    )(x, Wcm, welp, werp)


EB2 = 64
NB2 = E // EB2


def _layer0_body(featc_hbm, elt_hbm, ert_hbm, src_hbm, dst_hbm,
                 exmat_out, den_out, rst_out,
                 sidxA0, didxA0, sidxB0, didxB0, sidxA1, didxA1, sidxB1,
                 didxB1, bufA, bufB, exrA, exrB, zb, rst_sp,
                 semA, semB, semXA, semXB, semIA0, semIB0, semIA1, semIB1):
    cid, sid, wid = _worker_id()
    base_b, nb = _batch_range(sid, NB2, NS)

    _zero_rows(zb, 16, 128)
    _zero_spmem(rst_sp, zb, sid)
    plsc.subcore_barrier()

    def body0(j, _):
        base = (base_b + j) * EB2
        pltpu.sync_copy(src_hbm.at[pl.ds(base, EB2)], sidxA0)
        pltpu.sync_copy(dst_hbm.at[pl.ds(base, EB2)], didxA0)
        cp1 = pltpu.async_copy(elt_hbm.at[sidxA0], bufA, semA)
        cp2 = pltpu.async_copy(ert_hbm.at[didxA0], bufB, semB)
        cp1.wait()
        cp2.wait()

        def ebody(i, _):
            v = bufA[i, pl.ds(0, L)] + bufB[i, pl.ds(0, L)]
            v = jnp.where(v >= 0, v, v * NEG_SLOPE)
            ex = jnp.exp(v)
            exrA[i, :] = ex
            bufA[i, pl.ds(0, L)] = ex
            return 0
        lax.fori_loop(0, EB2, ebody, 0, unroll=4)

        pltpu.sync_copy(exrA, exmat_out.at[cid, pl.ds(base, EB2)])

        @pl.when(cid == 0)
        def _():
            pltpu.sync_copy(bufA, rst_sp.at[didxA0], add=True)
        return 0

    lax.fori_loop(0, nb, body0, 0)
    plsc.subcore_barrier()

    @pl.when(cid == 0)
    def _():
        _copyout_spmem(rst_sp, den_out, sid)
    plsc.subcore_barrier()

    for ci in range(N_CHUNKS // NC):
        c = 2 * ci + cid
        coff = c * N
        _zero_spmem(rst_sp, zb, sid)
        plsc.subcore_barrier()

        def pref_idx(j, sidx, didx, semI):
            base = (base_b + j) * EB2
            pltpu.async_copy(src_hbm.at[pl.ds(base, EB2)], sidx, semI)
            pltpu.async_copy(dst_hbm.at[pl.ds(base, EB2)], didx, semI)

        def launch(j, sidx, didx, buf, exr, sem, semX, semI):
            base = (base_b + j) * EB2
            pltpu.make_async_copy(
                src_hbm.at[pl.ds(base, EB2)], sidx, semI).wait()
            pltpu.make_async_copy(
                dst_hbm.at[pl.ds(base, EB2)], didx, semI).wait()
            for g in range(EB2 // L):
                sidx[pl.ds(g * L, L)] = sidx[pl.ds(g * L, L)] + coff
            pltpu.async_copy(featc_hbm.at[sidx], buf, sem)
            pltpu.async_copy(exmat_out.at[cid, pl.ds(base, EB2)], exr, semX)

        def waitbufs(j, sidx, buf, exr, sem, semX):
            base = (base_b + j) * EB2
            pltpu.make_async_copy(featc_hbm.at[sidx], buf, sem).wait()
            pltpu.make_async_copy(
                exmat_out.at[cid, pl.ds(base, EB2)], exr, semX).wait()

        def compute(didx, buf, exr):
            def rbody(i, _):
                s = exr[i, :][1 + ci]
                for jv in range(CHUNK // L):
                    buf[i, pl.ds(jv * L, L)] = buf[i, pl.ds(jv * L, L)] * s
                return 0
            lax.fori_loop(0, EB2, rbody, 0, unroll=4)
            pltpu.sync_copy(buf, rst_sp.at[didx], add=True)

        isets = [(sidxA0, didxA0, semIA0), (sidxB0, didxB0, semIB0),
                 (sidxA1, didxA1, semIA1), (sidxB1, didxB1, semIB1)]
        dsets = [(bufA, exrA, semA, semXA), (bufB, exrB, semB, semXB)]

        def prefq(j, q):
            @pl.when(j < nb)
            def _():
                pref_idx(j, *isets[q])

        def launchq(j, q):
            @pl.when(j < nb)
            def _():
                si, di, smi = isets[q]
                bf, xr, sm, smx = dsets[q % 2]
                launch(j, si, di, bf, xr, sm, smx, smi)

        for q in range(4):
            prefq(q, q)
        launchq(0, 0)
        launchq(1, 1)

        def quad(p, _):
            jA = 4 * p
            for q in range(4):
                j = jA + q
                si, di, smi = isets[q]
                bf, xr, sm, smx = dsets[q % 2]
                waitbufs(j, si, bf, xr, sm, smx)
                compute(di, bf, xr)
                prefq(j + 4, q)
                launchq(j + 2, (q + 2) % 4)
            return 0

        lax.fori_loop(0, nb // 4, quad, 0)

        @pl.when(nb % 4 == 1)
        def _():
            waitbufs(nb - 1, sidxA0, bufA, exrA, semA, semXA)
            compute(didxA0, bufA, exrA)

        plsc.subcore_barrier()
        _copyout_spmem(rst_sp, rst_out.at[c], sid)
        plsc.subcore_barrier()


def _layer0_sc(featc, elt, ert, src, dst):
    f = pl.kernel(
        _layer0_body,
        out_type=[
            jax.ShapeDtypeStruct((NC, E, 16), jnp.float32),
            jax.ShapeDtypeStruct((N, 128), jnp.float32),
            jax.ShapeDtypeStruct((N_CHUNKS, N, CHUNK), jnp.float32),
        ],
        mesh=_sc_mesh(),
        scratch_types=(
            [pltpu.VMEM((EB2,), jnp.int32)] * 8
            + [pltpu.VMEM((EB2, CHUNK), jnp.float32)] * 2
            + [pltpu.VMEM((EB2, 16), jnp.float32)] * 2
            + [pltpu.VMEM((16, 128), jnp.float32),
               pltpu.VMEM_SHARED((N, CHUNK), jnp.float32)]
            + [pltpu.SemaphoreType.DMA] * 8
        ),
    )
    return f(featc, elt, ert, src, dst)


def _projD_body(rst_ref, den_ref, oh_ref, b0_ref, w1_ref, wel_ref, wer_ref,
                wres_ref, feat1_ref, elt1_ref, ert1_ref, res_ref):
    c = pl.program_id(1)
    rst = rst_ref[0]
    den = jnp.dot(den_ref[...], oh_ref[0],
                  preferred_element_type=jnp.float32)
    hc = rst / jnp.maximum(den, 1e-9) + b0_ref[0]
    hc = jnp.where(hc > 0, hc, jnp.exp(hc) - 1.0)

    f1 = jnp.dot(hc, w1_ref[0], preferred_element_type=jnp.float32)
    l1 = jnp.dot(hc, wel_ref[0], preferred_element_type=jnp.float32)
    e1 = jnp.dot(hc, wer_ref[0], preferred_element_type=jnp.float32)
    r1 = jnp.dot(hc, wres_ref[0], preferred_element_type=jnp.float32)

    @pl.when(c == 0)
    def _():
        feat1_ref[...] = f1
        elt1_ref[...] = l1
        ert1_ref[...] = e1
        res_ref[...] = r1

    @pl.when(c != 0)
    def _():
        feat1_ref[...] += f1
        elt1_ref[...] += l1
        ert1_ref[...] += e1
        res_ref[...] += r1


def _projD(rstU, den, onehots, b0cm, W1cm, Wel1cm, Wer1cm, resWcm):
    grid = (N // ROW_BLK, N_CHUNKS)
    return pl.pallas_call(
        _projD_body,
        grid=grid,
        in_specs=[
            pl.BlockSpec((1, ROW_BLK, CHUNK), lambda i, c: (c, i, 0)),
            pl.BlockSpec((ROW_BLK, 128), lambda i, c: (i, 0)),
            pl.BlockSpec((1, 128, 1), lambda i, c: (c, 0, 0)),
            pl.BlockSpec((1, 1, CHUNK), lambda i, c: (c, 0, 0)),
            pl.BlockSpec((1, CHUNK, 128), lambda i, c: (c, 0, 0)),
            pl.BlockSpec((1, CHUNK, 128), lambda i, c: (c, 0, 0)),
            pl.BlockSpec((1, CHUNK, 128), lambda i, c: (c, 0, 0)),
            pl.BlockSpec((1, CHUNK, 128), lambda i, c: (c, 0, 0)),
        ],
        out_specs=[
            pl.BlockSpec((ROW_BLK, 128), lambda i, c: (i, 0)),
            pl.BlockSpec((ROW_BLK, 128), lambda i, c: (i, 0)),
            pl.BlockSpec((ROW_BLK, 128), lambda i, c: (i, 0)),
            pl.BlockSpec((ROW_BLK, 128), lambda i, c: (i, 0)),
        ],
        out_shape=[
            jax.ShapeDtypeStruct((N, 128), jnp.float32),
            jax.ShapeDtypeStruct((N, 128), jnp.float32),
            jax.ShapeDtypeStruct((N, 128), jnp.float32),
            jax.ShapeDtypeStruct((N, 128), jnp.float32),
        ],
    )(rstU, den, onehots, b0cm, W1cm, Wel1cm, Wer1cm, resWcm)


NH = N // NC
NHP = NH + 8
RS2 = 312


def _zero_spmem_h(sp_ref, zb, sid):
    def body(k, _):
        pltpu.sync_copy(
            zb, sp_ref.at[pl.ds(pl.multiple_of((sid * 40 + k) * 8, 8), 8)])
        return 0
    cnt = jnp.minimum(40, jnp.maximum(NHP // 8 - sid * 40, 0))
    lax.fori_loop(0, cnt, body, 0)


def _copyout_spmem_h(sp_ref, out_ref, cid, sid):
    r0 = pl.multiple_of(sid * RS2, 8)
    o0 = pl.multiple_of(NH * cid + sid * RS2, 8)
    pltpu.sync_copy(sp_ref.at[pl.ds(r0, RS2)], out_ref.at[pl.ds(o0, RS2)])

    @pl.when(sid == NS - 1)
    def _():
        pltpu.sync_copy(
            sp_ref.at[pl.ds(NS * RS2, NH - NS * RS2)],
            out_ref.at[pl.ds(pl.multiple_of(NH * cid + NS * RS2, 8),
                             NH - NS * RS2)])


def _edge1_body(f1_hbm, elt_hbm, ert_hbm, src_hbm, dst_hbm,
                den_out, rst_out,
                sidxA0, didxA0, sidxB0, didxB0, sidxA1, didxA1, sidxB1,
                didxB1, didx2, fA, sA, dA, fB, sB, dB, exwide, zb, acc_sp,
                semA, semB, semIA0, semIB0, semIA1, semIB1):
    cid, sid, wid = _worker_id()
    base_b, nb = _batch_range(sid, NB, NS)
    off = cid * NH

    _zero_rows(zb, 8, 128)
    _zero_rows(exwide, EB, 128)
    _zero_spmem_h(acc_sp, zb, sid)
    plsc.subcore_barrier()

    isets = [(sidxA0, didxA0, semIA0), (sidxB0, didxB0, semIB0),
             (sidxA1, didxA1, semIA1), (sidxB1, didxB1, semIB1)]

    def pref_idx(j, sidx, didx, semI):
        base = (base_b + j) * EB
        pltpu.async_copy(src_hbm.at[pl.ds(base, EB)], sidx, semI)
        pltpu.async_copy(dst_hbm.at[pl.ds(base, EB)], didx, semI)

    def wait_idx(j, sidx, didx, semI):
        base = (base_b + j) * EB
        pltpu.make_async_copy(src_hbm.at[pl.ds(base, EB)], sidx, semI).wait()
        pltpu.make_async_copy(dst_hbm.at[pl.ds(base, EB)], didx, semI).wait()

    def mk_didx2(didx):
        for g in range(EB // L):
            dl = didx[pl.ds(g * L, L)] - off
            ok = (dl >= 0) & (dl < NH)
            didx2[pl.ds(g * L, L)] = jnp.where(ok, dl, NH)

    def run_phase(launch, compute):
        def prefq(j, q):
            @pl.when(j < nb)
            def _():
                pref_idx(j, *isets[q])

        def launchq(j, q):
            @pl.when(j < nb)
            def _():
                launch(j, q)

        for q in range(4):
            prefq(q, q)
        launchq(0, 0)
        launchq(1, 1)

        def quad(p, _):
            jA = 4 * p
            for q in range(4):
                j = jA + q
                compute(j, q)
                prefq(j + 4, q)
                launchq(j + 2, (q + 2) % 4)
            return 0

        lax.fori_loop(0, nb // 4, quad, 0)

        @pl.when(nb % 4 == 1)
        def _():
            compute(nb - 1, 0)

    def launch1(j, q):
        si, di, smi = isets[q]
        sbuf, dbuf, sem = (sA, dA, semA) if q % 2 == 0 else (sB, dB, semB)
        wait_idx(j, si, di, smi)
        pltpu.async_copy(elt_hbm.at[si], sbuf, sem)
        pltpu.async_copy(ert_hbm.at[di], dbuf, sem)

    def compute1(j, q):
        si, di, smi = isets[q]
        sbuf, dbuf, sem = (sA, dA, semA) if q % 2 == 0 else (sB, dB, semB)
        pltpu.make_async_copy(elt_hbm.at[si], sbuf, sem).wait()
        pltpu.make_async_copy(ert_hbm.at[di], dbuf, sem).wait()

        def ebody(i, _):
            v = sbuf[i, pl.ds(0, L)] + dbuf[i, pl.ds(0, L)]
            v = jnp.where(v >= 0, v, v * NEG_SLOPE)
            ex = jnp.exp(v)
            exwide[i, pl.ds(0, L)] = ex
            return 0
        lax.fori_loop(0, EB, ebody, 0, unroll=4)
        mk_didx2(di)
        pltpu.sync_copy(exwide, acc_sp.at[didx2], add=True)

    run_phase(launch1, compute1)
    plsc.subcore_barrier()
    _copyout_spmem_h(acc_sp, den_out, cid, sid)
    plsc.subcore_barrier()
    _zero_spmem_h(acc_sp, zb, sid)
    plsc.subcore_barrier()

    def launch2(j, q):
        si, di, smi = isets[q]
        fbuf, sbuf, dbuf, sem = (
            (fA, sA, dA, semA) if q % 2 == 0 else (fB, sB, dB, semB))
        wait_idx(j, si, di, smi)
        pltpu.async_copy(f1_hbm.at[si], fbuf, sem)
        pltpu.async_copy(elt_hbm.at[si], sbuf, sem)
        pltpu.async_copy(ert_hbm.at[di], dbuf, sem)

    def compute2(j, q):
        si, di, smi = isets[q]
        fbuf, sbuf, dbuf, sem = (
            (fA, sA, dA, semA) if q % 2 == 0 else (fB, sB, dB, semB))
        pltpu.make_async_copy(f1_hbm.at[si], fbuf, sem).wait()
        pltpu.make_async_copy(elt_hbm.at[si], sbuf, sem).wait()
        pltpu.make_async_copy(ert_hbm.at[di], dbuf, sem).wait()

        def rbody(i, _):
            v = sbuf[i, pl.ds(0, L)] + dbuf[i, pl.ds(0, L)]
            v = jnp.where(v >= 0, v, v * NEG_SLOPE)
            s = jnp.exp(v)[1]
            for jv in range(128 // L):
                fbuf[i, pl.ds(jv * L, L)] = fbuf[i, pl.ds(jv * L, L)] * s
            return 0
        lax.fori_loop(0, EB, rbody, 0, unroll=4)
        mk_didx2(di)
        pltpu.sync_copy(fbuf, acc_sp.at[didx2], add=True)

    run_phase(launch2, compute2)
    plsc.subcore_barrier()
    _copyout_spmem_h(acc_sp, rst_out, cid, sid)


def _edge1(feat1, elt1, ert1, src, dst):
    f = pl.kernel(
        _edge1_body,
        out_type=[
            jax.ShapeDtypeStruct((N, 128), jnp.float32),
            jax.ShapeDtypeStruct((N, 128), jnp.float32),
        ],
        mesh=_sc_mesh(),
        scratch_types=(
            [pltpu.VMEM((EB,), jnp.int32)] * 9
            + [pltpu.VMEM((EB, 128), jnp.float32)] * 7
            + [pltpu.VMEM((8, 128), jnp.float32),
               pltpu.VMEM_SHARED((NHP, 128), jnp.float32)]
            + [pltpu.SemaphoreType.DMA] * 6
        ),
    )
    return f(feat1, elt1, ert1, src, dst)


def _finF_body(rst_ref, den_ref, res_ref, b1_ref, out_ref):
    num = rst_ref[...]
    den = den_ref[:, 1:2]
    out_ref[...] = num / jnp.maximum(den, 1e-9) + res_ref[...] + b1_ref[...]


def _finF(rst1, den1, res, b1):
    grid = (N // ROW_BLK,)
    return pl.pallas_call(
        _finF_body,
        grid=grid,
        in_specs=[
            pl.BlockSpec((ROW_BLK, 128), lambda i: (i, 0)),
            pl.BlockSpec((ROW_BLK, 128), lambda i: (i, 0)),
            pl.BlockSpec((ROW_BLK, 128), lambda i: (i, 0)),
            pl.BlockSpec((1, 128), lambda i: (0, 0)),
        ],
        out_specs=pl.BlockSpec((ROW_BLK, 128), lambda i: (i, 0)),
        out_shape=jax.ShapeDtypeStruct((N, 128), jnp.float32),
    )(rst1, den1, res, b1)


def kernel(x, edge_index, W0, al0, ar0, b0, W1, al1, ar1, b1, resW):
    src = edge_index[0]
    dst = edge_index[1]

    W0h = W0.reshape(IN_DIM, H0, HID)
    wel0 = jnp.einsum("khd,hd->kh", W0h, al0)
    wer0 = jnp.einsum("khd,hd->kh", W0h, ar0)
    zc = jnp.zeros((IN_DIM, 1), jnp.float32)
    zpad = jnp.zeros((IN_DIM, 128 - 9), jnp.float32)
    welp = jnp.concatenate([zc, wel0, zpad], axis=1)
    werp = jnp.concatenate([zc, wer0, zpad], axis=1)
    Wcm = W0.reshape(IN_DIM, N_CHUNKS, CHUNK).transpose(1, 0, 2)

    featc, elt, ert = _projA(x, Wcm, welp, werp)
    exmat, den0, rstU = _layer0_sc(featc, elt, ert, src, dst)
    del exmat

    wel1 = W1 @ al1[0]
    wer1 = W1 @ ar1[0]
    W1cm = W1.reshape(N_CHUNKS, CHUNK, NUM_CLASSES)
    z16 = jnp.zeros((N_CHUNKS, CHUNK, 1), jnp.float32)
    zw = jnp.zeros((N_CHUNKS, CHUNK, 126), jnp.float32)
    Wel1cm = jnp.concatenate(
        [z16, wel1.reshape(N_CHUNKS, CHUNK, 1), zw], axis=2)
    Wer1cm = jnp.concatenate(
        [z16, wer1.reshape(N_CHUNKS, CHUNK, 1), zw], axis=2)
    resWcm = resW.reshape(N_CHUNKS, CHUNK, NUM_CLASSES)
    b0cm = b0.reshape(N_CHUNKS, 1, CHUNK)
    heads = jnp.arange(N_CHUNKS) // 2
    onehots = (jnp.arange(128)[None, :, None] == (1 + heads)[:, None, None])
    onehots = onehots.astype(jnp.float32)

    feat1, elt1, ert1, res = _projD(
        rstU, den0, onehots, b0cm, W1cm, Wel1cm, Wer1cm, resWcm)

    den1, rst1 = _edge1(feat1, elt1, ert1, src, dst)

    return _finF(rst1, den1, res, b1[None, :])


# speedup vs baseline: 1.1257x; 1.1257x over previous
import functools

import jax
import jax.numpy as jnp
from jax import lax
from jax.experimental import pallas as pl
from jax.experimental.pallas import tpu as pltpu
from jax.experimental.pallas import tpu_sc as plsc

N = 10000
E = 160000
IN_DIM = 256
HID = 256
H0 = 8
NUM_CLASSES = 128
NEG_SLOPE = 0.2

ROW_BLK = 1000
N_CHUNKS = 16
CHUNK = 128

NC = 2
NS = 16
L = 16
NW = NC * NS
EB = 64
NB = E // EB
RS = 624


def _sc_mesh():
    return plsc.VectorSubcoreMesh(core_axis_name="c", subcore_axis_name="s")


def _worker_id():
    sid = lax.axis_index("s")
    cid = lax.axis_index("c")
    return cid, sid, sid * NC + cid


def _batch_range(wid, nb_total, nw):
    per = nb_total // nw
    extra = nb_total % nw
    base = wid * per + jnp.minimum(wid, extra)
    cnt = per + (wid < extra).astype(jnp.int32)
    return base, cnt


def _zero_rows(ref, n_rows, width):
    def body(i, _):
        for j in range(width // L):
            ref[i, pl.ds(j * L, L)] = jnp.zeros((L,), jnp.float32)
        return 0
    lax.fori_loop(0, n_rows, body, 0)


def _zero_spmem(sp_ref, zb, sid):
    r0 = sid * RS

    def body(k, _):
        pltpu.sync_copy(zb, sp_ref.at[pl.ds(pl.multiple_of(r0 + k * 16, 16), 16)])
        return 0
    lax.fori_loop(0, RS // 16 + (sid == NS - 1).astype(jnp.int32), body, 0)


def _copyout_spmem(sp_ref, out_ref, sid):
    r0 = pl.multiple_of(sid * RS, 16)
    pltpu.sync_copy(sp_ref.at[pl.ds(r0, RS)], out_ref.at[pl.ds(r0, RS)])

    @pl.when(sid == NS - 1)
    def _():
        pltpu.sync_copy(sp_ref.at[pl.ds(NS * RS, N - NS * RS)],
                        out_ref.at[pl.ds(NS * RS, N - NS * RS)])


def _projA_body(x_ref, w_ref, welp_ref, werp_ref, feat_ref, elt_ref, ert_ref):
    x = x_ref[...]
    feat_ref[...] = jnp.dot(x, w_ref[0], preferred_element_type=jnp.float32)
    elt_ref[...] = jnp.dot(x, welp_ref[...], preferred_element_type=jnp.float32)
    ert_ref[...] = jnp.dot(x, werp_ref[...], preferred_element_type=jnp.float32)


def _projA(x, Wcm, welp, werp):
    grid = (N // ROW_BLK, N_CHUNKS)
    return pl.pallas_call(
        _projA_body,
        grid=grid,
        in_specs=[
            pl.BlockSpec((ROW_BLK, IN_DIM), lambda i, c: (i, 0)),
            pl.BlockSpec((1, IN_DIM, CHUNK), lambda i, c: (c, 0, 0)),
            pl.BlockSpec((IN_DIM, 128), lambda i, c: (0, 0)),
            pl.BlockSpec((IN_DIM, 128), lambda i, c: (0, 0)),
        ],
        out_specs=[
            pl.BlockSpec((ROW_BLK, CHUNK),
                         lambda i, c: (c * (N // ROW_BLK) + i, 0)),
            pl.BlockSpec((ROW_BLK, 128), lambda i, c: (i, 0)),
            pl.BlockSpec((ROW_BLK, 128), lambda i, c: (i, 0)),
        ],
        out_shape=[
            jax.ShapeDtypeStruct((N_CHUNKS * N, CHUNK), jnp.float32),
            jax.ShapeDtypeStruct((N, 128), jnp.float32),
            jax.ShapeDtypeStruct((N, 128), jnp.float32),
        ],
    )(x, Wcm, welp, werp)


EB2 = 80
NB2 = E // EB2


def _layer0_body(featc_hbm, elt_hbm, ert_hbm, src_hbm, dst_hbm,
                 exmat_out, den_out, rst_out,
                 sidxA0, didxA0, sidxB0, didxB0, sidxA1, didxA1, sidxB1,
                 didxB1, bufA, bufB, exrA, exrB, zb, rst_sp,
                 semA, semB, semXA, semXB, semIA0, semIB0, semIA1, semIB1):
    cid, sid, wid = _worker_id()
    base_b, nb = _batch_range(sid, NB2, NS)

    _zero_rows(zb, 16, 128)
    _zero_spmem(rst_sp, zb, sid)
    plsc.subcore_barrier()

    def body0(j, _):
        base = (base_b + j) * EB2
        pltpu.sync_copy(src_hbm.at[pl.ds(base, EB2)], sidxA0)
        pltpu.sync_copy(dst_hbm.at[pl.ds(base, EB2)], didxA0)
        cp1 = pltpu.async_copy(elt_hbm.at[sidxA0], bufA, semA)
        cp2 = pltpu.async_copy(ert_hbm.at[didxA0], bufB, semB)
        cp1.wait()
        cp2.wait()

        def ebody(i, _):
            v = bufA[i, pl.ds(0, L)] + bufB[i, pl.ds(0, L)]
            v = jnp.where(v >= 0, v, v * NEG_SLOPE)
            ex = jnp.exp(v)
            exrA[i, :] = ex
            bufA[i, pl.ds(0, L)] = ex
            return 0
        lax.fori_loop(0, EB2, ebody, 0)

        pltpu.sync_copy(exrA, exmat_out.at[cid, pl.ds(base, EB2)])

        @pl.when(cid == 0)
        def _():
            pltpu.sync_copy(bufA, rst_sp.at[didxA0], add=True)
        return 0

    lax.fori_loop(0, nb, body0, 0)
    plsc.subcore_barrier()

    @pl.when(cid == 0)
    def _():
        _copyout_spmem(rst_sp, den_out, sid)
    plsc.subcore_barrier()

    for ci in range(N_CHUNKS // NC):
        c = 2 * ci + cid
        coff = c * N
        _zero_spmem(rst_sp, zb, sid)
        plsc.subcore_barrier()

        def pref_idx(j, sidx, didx, semI):
            base = (base_b + j) * EB2
            pltpu.async_copy(src_hbm.at[pl.ds(base, EB2)], sidx, semI)
            pltpu.async_copy(dst_hbm.at[pl.ds(base, EB2)], didx, semI)

        def launch(j, sidx, didx, buf, exr, sem, semX, semI):
            base = (base_b + j) * EB2
            pltpu.make_async_copy(
                src_hbm.at[pl.ds(base, EB2)], sidx, semI).wait()
            pltpu.make_async_copy(
                dst_hbm.at[pl.ds(base, EB2)], didx, semI).wait()
            for g in range(EB2 // L):
                sidx[pl.ds(g * L, L)] = sidx[pl.ds(g * L, L)] + coff
            pltpu.async_copy(featc_hbm.at[sidx], buf, sem)
            pltpu.async_copy(exmat_out.at[cid, pl.ds(base, EB2)], exr, semX)

        def waitbufs(j, sidx, buf, exr, sem, semX):
            base = (base_b + j) * EB2
            pltpu.make_async_copy(featc_hbm.at[sidx], buf, sem).wait()
            pltpu.make_async_copy(
                exmat_out.at[cid, pl.ds(base, EB2)], exr, semX).wait()

        def compute(didx, buf, exr):
            def rbody(i, _):
                s = exr[i, :][1 + ci]
                for jv in range(CHUNK // L):
                    buf[i, pl.ds(jv * L, L)] = buf[i, pl.ds(jv * L, L)] * s
                return 0
            lax.fori_loop(0, EB2, rbody, 0)
            pltpu.sync_copy(buf, rst_sp.at[didx], add=True)

        isets = [(sidxA0, didxA0, semIA0), (sidxB0, didxB0, semIB0),
                 (sidxA1, didxA1, semIA1), (sidxB1, didxB1, semIB1)]
        dsets = [(bufA, exrA, semA, semXA), (bufB, exrB, semB, semXB)]

        def prefq(j, q):
            @pl.when(j < nb)
            def _():
                pref_idx(j, *isets[q])

        def launchq(j, q):
            @pl.when(j < nb)
            def _():
                si, di, smi = isets[q]
                bf, xr, sm, smx = dsets[q % 2]
                launch(j, si, di, bf, xr, sm, smx, smi)

        for q in range(4):
            prefq(q, q)
        launchq(0, 0)
        launchq(1, 1)

        def quad(p, _):
            jA = 4 * p
            for q in range(4):
                j = jA + q
                si, di, smi = isets[q]
                bf, xr, sm, smx = dsets[q % 2]
                waitbufs(j, si, bf, xr, sm, smx)
                compute(di, bf, xr)
                prefq(j + 4, q)
                launchq(j + 2, (q + 2) % 4)
            return 0

        lax.fori_loop(0, nb // 4, quad, 0)

        @pl.when(nb % 4 == 1)
        def _():
            waitbufs(nb - 1, sidxA0, bufA, exrA, semA, semXA)
            compute(didxA0, bufA, exrA)

        plsc.subcore_barrier()
        _copyout_spmem(rst_sp, rst_out.at[c], sid)
        plsc.subcore_barrier()


def _layer0_sc(featc, elt, ert, src, dst):
    f = pl.kernel(
        _layer0_body,
        out_type=[
            jax.ShapeDtypeStruct((NC, E, 16), jnp.float32),
            jax.ShapeDtypeStruct((N, 128), jnp.float32),
            jax.ShapeDtypeStruct((N_CHUNKS, N, CHUNK), jnp.float32),
        ],
        mesh=_sc_mesh(),
        scratch_types=(
            [pltpu.VMEM((EB2,), jnp.int32)] * 8
            + [pltpu.VMEM((EB2, CHUNK), jnp.float32)] * 2
            + [pltpu.VMEM((EB2, 16), jnp.float32)] * 2
            + [pltpu.VMEM((16, 128), jnp.float32),
               pltpu.VMEM_SHARED((N, CHUNK), jnp.float32)]
            + [pltpu.SemaphoreType.DMA] * 8
        ),
    )
    return f(featc, elt, ert, src, dst)


def _projD_body(rst_ref, den_ref, oh_ref, b0_ref, w1_ref, wel_ref, wer_ref,
                wres_ref, feat1_ref, elt1_ref, ert1_ref, res_ref):
    c = pl.program_id(1)
    rst = rst_ref[0]
    den = jnp.dot(den_ref[...], oh_ref[0],
                  preferred_element_type=jnp.float32)
    hc = rst / jnp.maximum(den, 1e-9) + b0_ref[0]
    hc = jnp.where(hc > 0, hc, jnp.exp(hc) - 1.0)

    f1 = jnp.dot(hc, w1_ref[0], preferred_element_type=jnp.float32)
    l1 = jnp.dot(hc, wel_ref[0], preferred_element_type=jnp.float32)
    e1 = jnp.dot(hc, wer_ref[0], preferred_element_type=jnp.float32)
    r1 = jnp.dot(hc, wres_ref[0], preferred_element_type=jnp.float32)

    @pl.when(c == 0)
    def _():
        feat1_ref[...] = f1
        elt1_ref[...] = l1
        ert1_ref[...] = e1
        res_ref[...] = r1

    @pl.when(c != 0)
    def _():
        feat1_ref[...] += f1
        elt1_ref[...] += l1
        ert1_ref[...] += e1
        res_ref[...] += r1


def _projD(rstU, den, onehots, b0cm, W1cm, Wel1cm, Wer1cm, resWcm):
    grid = (N // ROW_BLK, N_CHUNKS)
    return pl.pallas_call(
        _projD_body,
        grid=grid,
        in_specs=[
            pl.BlockSpec((1, ROW_BLK, CHUNK), lambda i, c: (c, i, 0)),
            pl.BlockSpec((ROW_BLK, 128), lambda i, c: (i, 0)),
            pl.BlockSpec((1, 128, 1), lambda i, c: (c, 0, 0)),
            pl.BlockSpec((1, 1, CHUNK), lambda i, c: (c, 0, 0)),
            pl.BlockSpec((1, CHUNK, 128), lambda i, c: (c, 0, 0)),
            pl.BlockSpec((1, CHUNK, 128), lambda i, c: (c, 0, 0)),
            pl.BlockSpec((1, CHUNK, 128), lambda i, c: (c, 0, 0)),
            pl.BlockSpec((1, CHUNK, 128), lambda i, c: (c, 0, 0)),
        ],
        out_specs=[
            pl.BlockSpec((ROW_BLK, 128), lambda i, c: (i, 0)),
            pl.BlockSpec((ROW_BLK, 128), lambda i, c: (i, 0)),
            pl.BlockSpec((ROW_BLK, 128), lambda i, c: (i, 0)),
            pl.BlockSpec((ROW_BLK, 128), lambda i, c: (i, 0)),
        ],
        out_shape=[
            jax.ShapeDtypeStruct((N, 128), jnp.float32),
            jax.ShapeDtypeStruct((N, 128), jnp.float32),
            jax.ShapeDtypeStruct((N, 128), jnp.float32),
            jax.ShapeDtypeStruct((N, 128), jnp.float32),
        ],
    )(rstU, den, onehots, b0cm, W1cm, Wel1cm, Wer1cm, resWcm)


NH = N // NC
NHP = NH + 8
RS2 = 312


def _zero_spmem_h(sp_ref, zb, sid):
    def body(k, _):
        pltpu.sync_copy(
            zb, sp_ref.at[pl.ds(pl.multiple_of((sid * 40 + k) * 8, 8), 8)])
        return 0
    cnt = jnp.minimum(40, jnp.maximum(NHP // 8 - sid * 40, 0))
    lax.fori_loop(0, cnt, body, 0)


def _copyout_spmem_h(sp_ref, out_ref, cid, sid):
    r0 = pl.multiple_of(sid * RS2, 8)
    o0 = pl.multiple_of(NH * cid + sid * RS2, 8)
    pltpu.sync_copy(sp_ref.at[pl.ds(r0, RS2)], out_ref.at[pl.ds(o0, RS2)])

    @pl.when(sid == NS - 1)
    def _():
        pltpu.sync_copy(
            sp_ref.at[pl.ds(NS * RS2, NH - NS * RS2)],
            out_ref.at[pl.ds(pl.multiple_of(NH * cid + NS * RS2, 8),
                             NH - NS * RS2)])


def _edge1_body(f1_hbm, elt_hbm, ert_hbm, src_hbm, dst_hbm,
                den_out, rst_out,
                sidxA0, didxA0, sidxB0, didxB0, sidxA1, didxA1, sidxB1,
                didxB1, didx2, fA, sA, dA, fB, sB, dB, exwide, zb, acc_sp,
                semA, semB, semIA0, semIB0, semIA1, semIB1):
    cid, sid, wid = _worker_id()
    base_b, nb = _batch_range(sid, NB, NS)
    off = cid * NH

    _zero_rows(zb, 8, 128)
    _zero_rows(exwide, EB, 128)
    _zero_spmem_h(acc_sp, zb, sid)
    plsc.subcore_barrier()

    isets = [(sidxA0, didxA0, semIA0), (sidxB0, didxB0, semIB0),
             (sidxA1, didxA1, semIA1), (sidxB1, didxB1, semIB1)]

    def pref_idx(j, sidx, didx, semI):
        base = (base_b + j) * EB
        pltpu.async_copy(src_hbm.at[pl.ds(base, EB)], sidx, semI)
        pltpu.async_copy(dst_hbm.at[pl.ds(base, EB)], didx, semI)

    def wait_idx(j, sidx, didx, semI):
        base = (base_b + j) * EB
        pltpu.make_async_copy(src_hbm.at[pl.ds(base, EB)], sidx, semI).wait()
        pltpu.make_async_copy(dst_hbm.at[pl.ds(base, EB)], didx, semI).wait()

    def mk_didx2(didx):
        for g in range(EB // L):
            dl = didx[pl.ds(g * L, L)] - off
            ok = (dl >= 0) & (dl < NH)
            didx2[pl.ds(g * L, L)] = jnp.where(ok, dl, NH)

    def run_phase(launch, compute):
        def prefq(j, q):
            @pl.when(j < nb)
            def _():
                pref_idx(j, *isets[q])

        def launchq(j, q):
            @pl.when(j < nb)
            def _():
                launch(j, q)

        for q in range(4):
            prefq(q, q)
        launchq(0, 0)
        launchq(1, 1)

        def quad(p, _):
            jA = 4 * p
            for q in range(4):
                j = jA + q
                compute(j, q)
                prefq(j + 4, q)
                launchq(j + 2, (q + 2) % 4)
            return 0

        lax.fori_loop(0, nb // 4, quad, 0)

        @pl.when(nb % 4 == 1)
        def _():
            compute(nb - 1, 0)

    def launch1(j, q):
        si, di, smi = isets[q]
        sbuf, dbuf, sem = (sA, dA, semA) if q % 2 == 0 else (sB, dB, semB)
        wait_idx(j, si, di, smi)
        pltpu.async_copy(elt_hbm.at[si], sbuf, sem)
        pltpu.async_copy(ert_hbm.at[di], dbuf, sem)

    def compute1(j, q):
        si, di, smi = isets[q]
        sbuf, dbuf, sem = (sA, dA, semA) if q % 2 == 0 else (sB, dB, semB)
        pltpu.make_async_copy(elt_hbm.at[si], sbuf, sem).wait()
        pltpu.make_async_copy(ert_hbm.at[di], dbuf, sem).wait()

        def ebody(i, _):
            v = sbuf[i, pl.ds(0, L)] + dbuf[i, pl.ds(0, L)]
            v = jnp.where(v >= 0, v, v * NEG_SLOPE)
            ex = jnp.exp(v)
            exwide[i, pl.ds(0, L)] = ex
            return 0
        lax.fori_loop(0, EB, ebody, 0)
        mk_didx2(di)
        pltpu.sync_copy(exwide, acc_sp.at[didx2], add=True)

    run_phase(launch1, compute1)
    plsc.subcore_barrier()
    _copyout_spmem_h(acc_sp, den_out, cid, sid)
    plsc.subcore_barrier()
    _zero_spmem_h(acc_sp, zb, sid)
    plsc.subcore_barrier()

    def launch2(j, q):
        si, di, smi = isets[q]
        fbuf, sbuf, dbuf, sem = (
            (fA, sA, dA, semA) if q % 2 == 0 else (fB, sB, dB, semB))
        wait_idx(j, si, di, smi)
        pltpu.async_copy(f1_hbm.at[si], fbuf, sem)
        pltpu.async_copy(elt_hbm.at[si], sbuf, sem)
        pltpu.async_copy(ert_hbm.at[di], dbuf, sem)

    def compute2(j, q):
        si, di, smi = isets[q]
        fbuf, sbuf, dbuf, sem = (
            (fA, sA, dA, semA) if q % 2 == 0 else (fB, sB, dB, semB))
        pltpu.make_async_copy(f1_hbm.at[si], fbuf, sem).wait()
        pltpu.make_async_copy(elt_hbm.at[si], sbuf, sem).wait()
        pltpu.make_async_copy(ert_hbm.at[di], dbuf, sem).wait()

        def rbody(i, _):
            v = sbuf[i, pl.ds(0, L)] + dbuf[i, pl.ds(0, L)]
            v = jnp.where(v >= 0, v, v * NEG_SLOPE)
            s = jnp.exp(v)[1]
            for jv in range(128 // L):
                fbuf[i, pl.ds(jv * L, L)] = fbuf[i, pl.ds(jv * L, L)] * s
            return 0
        lax.fori_loop(0, EB, rbody, 0)
        mk_didx2(di)
        pltpu.sync_copy(fbuf, acc_sp.at[didx2], add=True)

    run_phase(launch2, compute2)
    plsc.subcore_barrier()
    _copyout_spmem_h(acc_sp, rst_out, cid, sid)


def _edge1(feat1, elt1, ert1, src, dst):
    f = pl.kernel(
        _edge1_body,
        out_type=[
            jax.ShapeDtypeStruct((N, 128), jnp.float32),
            jax.ShapeDtypeStruct((N, 128), jnp.float32),
        ],
        mesh=_sc_mesh(),
        scratch_types=(
            [pltpu.VMEM((EB,), jnp.int32)] * 9
            + [pltpu.VMEM((EB, 128), jnp.float32)] * 7
            + [pltpu.VMEM((8, 128), jnp.float32),
               pltpu.VMEM_SHARED((NHP, 128), jnp.float32)]
            + [pltpu.SemaphoreType.DMA] * 6
        ),
    )
    return f(feat1, elt1, ert1, src, dst)


def _finF_body(rst_ref, den_ref, res_ref, b1_ref, out_ref):
    num = rst_ref[...]
    den = den_ref[:, 1:2]
    out_ref[...] = num / jnp.maximum(den, 1e-9) + res_ref[...] + b1_ref[...]


def _finF(rst1, den1, res, b1):
    grid = (N // ROW_BLK,)
    return pl.pallas_call(
        _finF_body,
        grid=grid,
        in_specs=[
            pl.BlockSpec((ROW_BLK, 128), lambda i: (i, 0)),
            pl.BlockSpec((ROW_BLK, 128), lambda i: (i, 0)),
            pl.BlockSpec((ROW_BLK, 128), lambda i: (i, 0)),
            pl.BlockSpec((1, 128), lambda i: (0, 0)),
        ],
        out_specs=pl.BlockSpec((ROW_BLK, 128), lambda i: (i, 0)),
        out_shape=jax.ShapeDtypeStruct((N, 128), jnp.float32),
    )(rst1, den1, res, b1)


def kernel(x, edge_index, W0, al0, ar0, b0, W1, al1, ar1, b1, resW):
    src = edge_index[0]
    dst = edge_index[1]

    W0h = W0.reshape(IN_DIM, H0, HID)
    wel0 = jnp.einsum("khd,hd->kh", W0h, al0)
    wer0 = jnp.einsum("khd,hd->kh", W0h, ar0)
    zc = jnp.zeros((IN_DIM, 1), jnp.float32)
    zpad = jnp.zeros((IN_DIM, 128 - 9), jnp.float32)
    welp = jnp.concatenate([zc, wel0, zpad], axis=1)
    werp = jnp.concatenate([zc, wer0, zpad], axis=1)
    Wcm = W0.reshape(IN_DIM, N_CHUNKS, CHUNK).transpose(1, 0, 2)

    featc, elt, ert = _projA(x, Wcm, welp, werp)
    exmat, den0, rstU = _layer0_sc(featc, elt, ert, src, dst)
    del exmat

    wel1 = W1 @ al1[0]
    wer1 = W1 @ ar1[0]
    W1cm = W1.reshape(N_CHUNKS, CHUNK, NUM_CLASSES)
    z16 = jnp.zeros((N_CHUNKS, CHUNK, 1), jnp.float32)
    zw = jnp.zeros((N_CHUNKS, CHUNK, 126), jnp.float32)
    Wel1cm = jnp.concatenate(
        [z16, wel1.reshape(N_CHUNKS, CHUNK, 1), zw], axis=2)
    Wer1cm = jnp.concatenate(
        [z16, wer1.reshape(N_CHUNKS, CHUNK, 1), zw], axis=2)
    resWcm = resW.reshape(N_CHUNKS, CHUNK, NUM_CLASSES)
    b0cm = b0.reshape(N_CHUNKS, 1, CHUNK)
    heads = jnp.arange(N_CHUNKS) // 2
    onehots = (jnp.arange(128)[None, :, None] == (1 + heads)[:, None, None])
    onehots = onehots.astype(jnp.float32)

    feat1, elt1, ert1, res = _projD(
        rstU, den0, onehots, b0cm, W1cm, Wel1cm, Wer1cm, resWcm)

    den1, rst1 = _edge1(feat1, elt1, ert1, src, dst)

    return _finF(rst1, den1, res, b1[None, :])


# speedup vs baseline: 1.1267x; 1.0009x over previous
import functools

import jax
import jax.numpy as jnp
from jax import lax
from jax.experimental import pallas as pl
from jax.experimental.pallas import tpu as pltpu
from jax.experimental.pallas import tpu_sc as plsc

N = 10000
E = 160000
IN_DIM = 256
HID = 256
H0 = 8
NUM_CLASSES = 128
NEG_SLOPE = 0.2

ROW_BLK = 1000
N_CHUNKS = 16
CHUNK = 128

NC = 2
NS = 16
L = 16
NW = NC * NS
EB = 80
NB = E // EB
RS = 624


def _sc_mesh():
    return plsc.VectorSubcoreMesh(core_axis_name="c", subcore_axis_name="s")


def _worker_id():
    sid = lax.axis_index("s")
    cid = lax.axis_index("c")
    return cid, sid, sid * NC + cid


def _batch_range(wid, nb_total, nw):
    per = nb_total // nw
    extra = nb_total % nw
    base = wid * per + jnp.minimum(wid, extra)
    cnt = per + (wid < extra).astype(jnp.int32)
    return base, cnt


def _zero_rows(ref, n_rows, width):
    def body(i, _):
        for j in range(width // L):
            ref[i, pl.ds(j * L, L)] = jnp.zeros((L,), jnp.float32)
        return 0
    lax.fori_loop(0, n_rows, body, 0)


def _zero_spmem(sp_ref, zb, sid):
    r0 = sid * RS

    def body(k, _):
        pltpu.sync_copy(zb, sp_ref.at[pl.ds(pl.multiple_of(r0 + k * 16, 16), 16)])
        return 0
    lax.fori_loop(0, RS // 16 + (sid == NS - 1).astype(jnp.int32), body, 0)


def _copyout_spmem(sp_ref, out_ref, sid):
    r0 = pl.multiple_of(sid * RS, 16)
    pltpu.sync_copy(sp_ref.at[pl.ds(r0, RS)], out_ref.at[pl.ds(r0, RS)])

    @pl.when(sid == NS - 1)
    def _():
        pltpu.sync_copy(sp_ref.at[pl.ds(NS * RS, N - NS * RS)],
                        out_ref.at[pl.ds(NS * RS, N - NS * RS)])


def _projA_body(x_ref, w_ref, welp_ref, werp_ref, feat_ref, elt_ref, ert_ref):
    x = x_ref[...]
    feat_ref[...] = jnp.dot(x, w_ref[0], preferred_element_type=jnp.float32)
    elt_ref[...] = jnp.dot(x, welp_ref[...], preferred_element_type=jnp.float32)
    ert_ref[...] = jnp.dot(x, werp_ref[...], preferred_element_type=jnp.float32)


def _projA(x, Wcm, welp, werp):
    grid = (N // ROW_BLK, N_CHUNKS)
    return pl.pallas_call(
        _projA_body,
        grid=grid,
        in_specs=[
            pl.BlockSpec((ROW_BLK, IN_DIM), lambda i, c: (i, 0)),
            pl.BlockSpec((1, IN_DIM, CHUNK), lambda i, c: (c, 0, 0)),
            pl.BlockSpec((IN_DIM, 128), lambda i, c: (0, 0)),
            pl.BlockSpec((IN_DIM, 128), lambda i, c: (0, 0)),
        ],
        out_specs=[
            pl.BlockSpec((ROW_BLK, CHUNK),
                         lambda i, c: (c * (N // ROW_BLK) + i, 0)),
            pl.BlockSpec((ROW_BLK, 128), lambda i, c: (i, 0)),
            pl.BlockSpec((ROW_BLK, 128), lambda i, c: (i, 0)),
        ],
        out_shape=[
            jax.ShapeDtypeStruct((N_CHUNKS * N, CHUNK), jnp.float32),
            jax.ShapeDtypeStruct((N, 128), jnp.float32),
            jax.ShapeDtypeStruct((N, 128), jnp.float32),
        ],
    )(x, Wcm, welp, werp)


EB2 = 80
NB2 = E // EB2


def _layer0_body(featc_hbm, elt_hbm, ert_hbm, src_hbm, dst_hbm,
                 exmat_out, den_out, rst_out,
                 sidxA0, didxA0, sidxB0, didxB0, sidxA1, didxA1, sidxB1,
                 didxB1, bufA, bufB, exrA, exrB, zb, rst_sp,
                 semA, semB, semXA, semXB, semIA0, semIB0, semIA1, semIB1):
    cid, sid, wid = _worker_id()
    base_b, nb = _batch_range(sid, NB2, NS)

    _zero_rows(zb, 16, 128)
    _zero_spmem(rst_sp, zb, sid)
    plsc.subcore_barrier()

    def body0(j, _):
        base = (base_b + j) * EB2
        pltpu.sync_copy(src_hbm.at[pl.ds(base, EB2)], sidxA0)
        pltpu.sync_copy(dst_hbm.at[pl.ds(base, EB2)], didxA0)
        cp1 = pltpu.async_copy(elt_hbm.at[sidxA0], bufA, semA)
        cp2 = pltpu.async_copy(ert_hbm.at[didxA0], bufB, semB)
        cp1.wait()
        cp2.wait()

        def ebody(i, _):
            v = bufA[i, pl.ds(0, L)] + bufB[i, pl.ds(0, L)]
            v = jnp.where(v >= 0, v, v * NEG_SLOPE)
            ex = jnp.exp(v)
            exrA[i, :] = ex
            bufA[i, pl.ds(0, L)] = ex
            return 0
        lax.fori_loop(0, EB2, ebody, 0)

        pltpu.sync_copy(exrA, exmat_out.at[cid, pl.ds(base, EB2)])

        @pl.when(cid == 0)
        def _():
            pltpu.sync_copy(bufA, rst_sp.at[didxA0], add=True)
        return 0

    lax.fori_loop(0, nb, body0, 0)
    plsc.subcore_barrier()

    @pl.when(cid == 0)
    def _():
        _copyout_spmem(rst_sp, den_out, sid)
    plsc.subcore_barrier()

    for ci in range(N_CHUNKS // NC):
        c = 2 * ci + cid
        coff = c * N
        _zero_spmem(rst_sp, zb, sid)
        plsc.subcore_barrier()

        def pref_idx(j, sidx, didx, semI):
            base = (base_b + j) * EB2
            pltpu.async_copy(src_hbm.at[pl.ds(base, EB2)], sidx, semI)
            pltpu.async_copy(dst_hbm.at[pl.ds(base, EB2)], didx, semI)

        def launch(j, sidx, didx, buf, exr, sem, semX, semI):
            base = (base_b + j) * EB2
            pltpu.make_async_copy(
                src_hbm.at[pl.ds(base, EB2)], sidx, semI).wait()
            pltpu.make_async_copy(
                dst_hbm.at[pl.ds(base, EB2)], didx, semI).wait()
            for g in range(EB2 // L):
                sidx[pl.ds(g * L, L)] = sidx[pl.ds(g * L, L)] + coff
            pltpu.async_copy(featc_hbm.at[sidx], buf, sem)
            pltpu.async_copy(exmat_out.at[cid, pl.ds(base, EB2)], exr, semX)

        def waitbufs(j, sidx, buf, exr, sem, semX):
            base = (base_b + j) * EB2
            pltpu.make_async_copy(featc_hbm.at[sidx], buf, sem).wait()
            pltpu.make_async_copy(
                exmat_out.at[cid, pl.ds(base, EB2)], exr, semX).wait()

        def compute(didx, buf, exr):
            def rbody(i, _):
                s = exr[i, :][1 + ci]
                for jv in range(CHUNK // L):
                    buf[i, pl.ds(jv * L, L)] = buf[i, pl.ds(jv * L, L)] * s
                return 0
            lax.fori_loop(0, EB2, rbody, 0)
            pltpu.sync_copy(buf, rst_sp.at[didx], add=True)

        isets = [(sidxA0, didxA0, semIA0), (sidxB0, didxB0, semIB0),
                 (sidxA1, didxA1, semIA1), (sidxB1, didxB1, semIB1)]
        dsets = [(bufA, exrA, semA, semXA), (bufB, exrB, semB, semXB)]

        def prefq(j, q):
            @pl.when(j < nb)
            def _():
                pref_idx(j, *isets[q])

        def launchq(j, q):
            @pl.when(j < nb)
            def _():
                si, di, smi = isets[q]
                bf, xr, sm, smx = dsets[q % 2]
                launch(j, si, di, bf, xr, sm, smx, smi)

        for q in range(4):
            prefq(q, q)
        launchq(0, 0)
        launchq(1, 1)

        def quad(p, _):
            jA = 4 * p
            for q in range(4):
                j = jA + q
                si, di, smi = isets[q]
                bf, xr, sm, smx = dsets[q % 2]
                waitbufs(j, si, bf, xr, sm, smx)
                compute(di, bf, xr)
                prefq(j + 4, q)
                launchq(j + 2, (q + 2) % 4)
            return 0

        lax.fori_loop(0, nb // 4, quad, 0)

        @pl.when(nb % 4 == 1)
        def _():
            waitbufs(nb - 1, sidxA0, bufA, exrA, semA, semXA)
            compute(didxA0, bufA, exrA)

        plsc.subcore_barrier()
        _copyout_spmem(rst_sp, rst_out.at[c], sid)
        plsc.subcore_barrier()


def _layer0_sc(featc, elt, ert, src, dst):
    f = pl.kernel(
        _layer0_body,
        out_type=[
            jax.ShapeDtypeStruct((NC, E, 16), jnp.float32),
            jax.ShapeDtypeStruct((N, 128), jnp.float32),
            jax.ShapeDtypeStruct((N_CHUNKS, N, CHUNK), jnp.float32),
        ],
        mesh=_sc_mesh(),
        scratch_types=(
            [pltpu.VMEM((EB2,), jnp.int32)] * 8
            + [pltpu.VMEM((EB2, CHUNK), jnp.float32)] * 2
            + [pltpu.VMEM((EB2, 16), jnp.float32)] * 2
            + [pltpu.VMEM((16, 128), jnp.float32),
               pltpu.VMEM_SHARED((N, CHUNK), jnp.float32)]
            + [pltpu.SemaphoreType.DMA] * 8
        ),
    )
    return f(featc, elt, ert, src, dst)


def _projD_body(rst_ref, den_ref, oh_ref, b0_ref, w1_ref, wel_ref, wer_ref,
                wres_ref, feat1_ref, elt1_ref, ert1_ref, res_ref):
    c = pl.program_id(1)
    rst = rst_ref[0]
    den = jnp.dot(den_ref[...], oh_ref[0],
                  preferred_element_type=jnp.float32)
    hc = rst / jnp.maximum(den, 1e-9) + b0_ref[0]
    hc = jnp.where(hc > 0, hc, jnp.exp(hc) - 1.0)

    f1 = jnp.dot(hc, w1_ref[0], preferred_element_type=jnp.float32)
    l1 = jnp.dot(hc, wel_ref[0], preferred_element_type=jnp.float32)
    e1 = jnp.dot(hc, wer_ref[0], preferred_element_type=jnp.float32)
    r1 = jnp.dot(hc, wres_ref[0], preferred_element_type=jnp.float32)

    @pl.when(c == 0)
    def _():
        feat1_ref[...] = f1
        elt1_ref[...] = l1
        ert1_ref[...] = e1
        res_ref[...] = r1

    @pl.when(c != 0)
    def _():
        feat1_ref[...] += f1
        elt1_ref[...] += l1
        ert1_ref[...] += e1
        res_ref[...] += r1


def _projD(rstU, den, onehots, b0cm, W1cm, Wel1cm, Wer1cm, resWcm):
    grid = (N // ROW_BLK, N_CHUNKS)
    return pl.pallas_call(
        _projD_body,
        grid=grid,
        in_specs=[
            pl.BlockSpec((1, ROW_BLK, CHUNK), lambda i, c: (c, i, 0)),
            pl.BlockSpec((ROW_BLK, 128), lambda i, c: (i, 0)),
            pl.BlockSpec((1, 128, 1), lambda i, c: (c, 0, 0)),
            pl.BlockSpec((1, 1, CHUNK), lambda i, c: (c, 0, 0)),
            pl.BlockSpec((1, CHUNK, 128), lambda i, c: (c, 0, 0)),
            pl.BlockSpec((1, CHUNK, 128), lambda i, c: (c, 0, 0)),
            pl.BlockSpec((1, CHUNK, 128), lambda i, c: (c, 0, 0)),
            pl.BlockSpec((1, CHUNK, 128), lambda i, c: (c, 0, 0)),
        ],
        out_specs=[
            pl.BlockSpec((ROW_BLK, 128), lambda i, c: (i, 0)),
            pl.BlockSpec((ROW_BLK, 128), lambda i, c: (i, 0)),
            pl.BlockSpec((ROW_BLK, 128), lambda i, c: (i, 0)),
            pl.BlockSpec((ROW_BLK, 128), lambda i, c: (i, 0)),
        ],
        out_shape=[
            jax.ShapeDtypeStruct((N, 128), jnp.float32),
            jax.ShapeDtypeStruct((N, 128), jnp.float32),
            jax.ShapeDtypeStruct((N, 128), jnp.float32),
            jax.ShapeDtypeStruct((N, 128), jnp.float32),
        ],
    )(rstU, den, onehots, b0cm, W1cm, Wel1cm, Wer1cm, resWcm)


NH = N // NC
NHP = NH + 8
RS2 = 312


def _zero_spmem_h(sp_ref, zb, sid):
    def body(k, _):
        pltpu.sync_copy(
            zb, sp_ref.at[pl.ds(pl.multiple_of((sid * 40 + k) * 8, 8), 8)])
        return 0
    cnt = jnp.minimum(40, jnp.maximum(NHP // 8 - sid * 40, 0))
    lax.fori_loop(0, cnt, body, 0)


def _copyout_spmem_h(sp_ref, out_ref, cid, sid):
    r0 = pl.multiple_of(sid * RS2, 8)
    o0 = pl.multiple_of(NH * cid + sid * RS2, 8)
    pltpu.sync_copy(sp_ref.at[pl.ds(r0, RS2)], out_ref.at[pl.ds(o0, RS2)])

    @pl.when(sid == NS - 1)
    def _():
        pltpu.sync_copy(
            sp_ref.at[pl.ds(NS * RS2, NH - NS * RS2)],
            out_ref.at[pl.ds(pl.multiple_of(NH * cid + NS * RS2, 8),
                             NH - NS * RS2)])


def _edge1_body(f1_hbm, elt_hbm, ert_hbm, src_hbm, dst_hbm,
                den_out, rst_out,
                sidxA0, didxA0, sidxB0, didxB0, sidxA1, didxA1, sidxB1,
                didxB1, didx2, fA, sA, dA, fB, sB, dB, exwide, zb, acc_sp,
                semA, semB, semIA0, semIB0, semIA1, semIB1):
    cid, sid, wid = _worker_id()
    base_b, nb = _batch_range(sid, NB, NS)
    off = cid * NH

    _zero_rows(zb, 8, 128)
    _zero_rows(exwide, EB, 128)
    _zero_spmem_h(acc_sp, zb, sid)
    plsc.subcore_barrier()

    isets = [(sidxA0, didxA0, semIA0), (sidxB0, didxB0, semIB0),
             (sidxA1, didxA1, semIA1), (sidxB1, didxB1, semIB1)]

    def pref_idx(j, sidx, didx, semI):
        base = (base_b + j) * EB
        pltpu.async_copy(src_hbm.at[pl.ds(base, EB)], sidx, semI)
        pltpu.async_copy(dst_hbm.at[pl.ds(base, EB)], didx, semI)

    def wait_idx(j, sidx, didx, semI):
        base = (base_b + j) * EB
        pltpu.make_async_copy(src_hbm.at[pl.ds(base, EB)], sidx, semI).wait()
        pltpu.make_async_copy(dst_hbm.at[pl.ds(base, EB)], didx, semI).wait()

    def mk_didx2(didx):
        for g in range(EB // L):
            dl = didx[pl.ds(g * L, L)] - off
            ok = (dl >= 0) & (dl < NH)
            didx2[pl.ds(g * L, L)] = jnp.where(ok, dl, NH)

    def run_phase(launch, compute):
        def prefq(j, q):
            @pl.when(j < nb)
            def _():
                pref_idx(j, *isets[q])

        def launchq(j, q):
            @pl.when(j < nb)
            def _():
                launch(j, q)

        for q in range(4):
            prefq(q, q)
        launchq(0, 0)
        launchq(1, 1)

        def quad(p, _):
            jA = 4 * p
            for q in range(4):
                j = jA + q
                compute(j, q)
                prefq(j + 4, q)
                launchq(j + 2, (q + 2) % 4)
            return 0

        lax.fori_loop(0, nb // 4, quad, 0)

        @pl.when(nb % 4 == 1)
        def _():
            compute(nb - 1, 0)

    def launch1(j, q):
        si, di, smi = isets[q]
        sbuf, dbuf, sem = (sA, dA, semA) if q % 2 == 0 else (sB, dB, semB)
        wait_idx(j, si, di, smi)
        pltpu.async_copy(elt_hbm.at[si], sbuf, sem)
        pltpu.async_copy(ert_hbm.at[di], dbuf, sem)

    def compute1(j, q):
        si, di, smi = isets[q]
        sbuf, dbuf, sem = (sA, dA, semA) if q % 2 == 0 else (sB, dB, semB)
        pltpu.make_async_copy(elt_hbm.at[si], sbuf, sem).wait()
        pltpu.make_async_copy(ert_hbm.at[di], dbuf, sem).wait()

        def ebody(i, _):
            v = sbuf[i, pl.ds(0, L)] + dbuf[i, pl.ds(0, L)]
            v = jnp.where(v >= 0, v, v * NEG_SLOPE)
            ex = jnp.exp(v)
            exwide[i, pl.ds(0, L)] = ex
            return 0
        lax.fori_loop(0, EB, ebody, 0)
        mk_didx2(di)
        pltpu.sync_copy(exwide, acc_sp.at[didx2], add=True)

    run_phase(launch1, compute1)
    plsc.subcore_barrier()
    _copyout_spmem_h(acc_sp, den_out, cid, sid)
    plsc.subcore_barrier()
    _zero_spmem_h(acc_sp, zb, sid)
    plsc.subcore_barrier()

    def launch2(j, q):
        si, di, smi = isets[q]
        fbuf, sbuf, dbuf, sem = (
            (fA, sA, dA, semA) if q % 2 == 0 else (fB, sB, dB, semB))
        wait_idx(j, si, di, smi)
        pltpu.async_copy(f1_hbm.at[si], fbuf, sem)
        pltpu.async_copy(elt_hbm.at[si], sbuf, sem)
        pltpu.async_copy(ert_hbm.at[di], dbuf, sem)

    def compute2(j, q):
        si, di, smi = isets[q]
        fbuf, sbuf, dbuf, sem = (
            (fA, sA, dA, semA) if q % 2 == 0 else (fB, sB, dB, semB))
        pltpu.make_async_copy(f1_hbm.at[si], fbuf, sem).wait()
        pltpu.make_async_copy(elt_hbm.at[si], sbuf, sem).wait()
        pltpu.make_async_copy(ert_hbm.at[di], dbuf, sem).wait()

        def rbody(i, _):
            v = sbuf[i, pl.ds(0, L)] + dbuf[i, pl.ds(0, L)]
            v = jnp.where(v >= 0, v, v * NEG_SLOPE)
            s = jnp.exp(v)[1]
            for jv in range(128 // L):
                fbuf[i, pl.ds(jv * L, L)] = fbuf[i, pl.ds(jv * L, L)] * s
            return 0
        lax.fori_loop(0, EB, rbody, 0)
        mk_didx2(di)
        pltpu.sync_copy(fbuf, acc_sp.at[didx2], add=True)

    run_phase(launch2, compute2)
    plsc.subcore_barrier()
    _copyout_spmem_h(acc_sp, rst_out, cid, sid)


def _edge1(feat1, elt1, ert1, src, dst):
    f = pl.kernel(
        _edge1_body,
        out_type=[
            jax.ShapeDtypeStruct((N, 128), jnp.float32),
            jax.ShapeDtypeStruct((N, 128), jnp.float32),
        ],
        mesh=_sc_mesh(),
        scratch_types=(
            [pltpu.VMEM((EB,), jnp.int32)] * 9
            + [pltpu.VMEM((EB, 128), jnp.float32)] * 7
            + [pltpu.VMEM((8, 128), jnp.float32),
               pltpu.VMEM_SHARED((NHP, 128), jnp.float32)]
            + [pltpu.SemaphoreType.DMA] * 6
        ),
    )
    return f(feat1, elt1, ert1, src, dst)


def _finF_body(rst_ref, den_ref, res_ref, b1_ref, out_ref):
    num = rst_ref[...]
    den = den_ref[:, 1:2]
    out_ref[...] = num / jnp.maximum(den, 1e-9) + res_ref[...] + b1_ref[...]


def _finF(rst1, den1, res, b1):
    grid = (N // ROW_BLK,)
    return pl.pallas_call(
        _finF_body,
        grid=grid,
        in_specs=[
            pl.BlockSpec((ROW_BLK, 128), lambda i: (i, 0)),
            pl.BlockSpec((ROW_BLK, 128), lambda i: (i, 0)),
            pl.BlockSpec((ROW_BLK, 128), lambda i: (i, 0)),
            pl.BlockSpec((1, 128), lambda i: (0, 0)),
        ],
        out_specs=pl.BlockSpec((ROW_BLK, 128), lambda i: (i, 0)),
        out_shape=jax.ShapeDtypeStruct((N, 128), jnp.float32),
    )(rst1, den1, res, b1)


def kernel(x, edge_index, W0, al0, ar0, b0, W1, al1, ar1, b1, resW):
    src = edge_index[0]
    dst = edge_index[1]

    W0h = W0.reshape(IN_DIM, H0, HID)
    wel0 = jnp.einsum("khd,hd->kh", W0h, al0)
    wer0 = jnp.einsum("khd,hd->kh", W0h, ar0)
    zc = jnp.zeros((IN_DIM, 1), jnp.float32)
    zpad = jnp.zeros((IN_DIM, 128 - 9), jnp.float32)
    welp = jnp.concatenate([zc, wel0, zpad], axis=1)
    werp = jnp.concatenate([zc, wer0, zpad], axis=1)
    Wcm = W0.reshape(IN_DIM, N_CHUNKS, CHUNK).transpose(1, 0, 2)

    featc, elt, ert = _projA(x, Wcm, welp, werp)
    exmat, den0, rstU = _layer0_sc(featc, elt, ert, src, dst)
    del exmat

    wel1 = W1 @ al1[0]
    wer1 = W1 @ ar1[0]
    W1cm = W1.reshape(N_CHUNKS, CHUNK, NUM_CLASSES)
    z16 = jnp.zeros((N_CHUNKS, CHUNK, 1), jnp.float32)
    zw = jnp.zeros((N_CHUNKS, CHUNK, 126), jnp.float32)
    Wel1cm = jnp.concatenate(
        [z16, wel1.reshape(N_CHUNKS, CHUNK, 1), zw], axis=2)
    Wer1cm = jnp.concatenate(
        [z16, wer1.reshape(N_CHUNKS, CHUNK, 1), zw], axis=2)
    resWcm = resW.reshape(N_CHUNKS, CHUNK, NUM_CLASSES)
    b0cm = b0.reshape(N_CHUNKS, 1, CHUNK)
    heads = jnp.arange(N_CHUNKS) // 2
    onehots = (jnp.arange(128)[None, :, None] == (1 + heads)[:, None, None])
    onehots = onehots.astype(jnp.float32)

    feat1, elt1, ert1, res = _projD(
        rstU, den0, onehots, b0cm, W1cm, Wel1cm, Wer1cm, resWcm)

    den1, rst1 = _edge1(feat1, elt1, ert1, src, dst)

    return _finF(rst1, den1, res, b1[None, :])
